# Initial kernel scaffold; baseline (speedup 1.0000x reference)
import jax
import jax.numpy as jnp
from jax.experimental import pallas as pl


def kernel(feature, edge_index, W1, b1, gamma, beta, W2, b2):
    raise NotImplementedError("write your pallas kernel here")


# speedup vs baseline: 4.3982x; 4.3982x over previous
import functools

import jax
import jax.numpy as jnp
from jax import lax
from jax.experimental import pallas as pl
from jax.experimental.pallas import tpu as pltpu
from jax.experimental.pallas import tpu_sc as plsc

EC = 64


def _rsqrt_newton(d):
    bits = plsc.bitcast(d, jnp.int32)
    bits = jnp.int32(0x5F3759DF) - (bits >> 1)
    y = plsc.bitcast(bits, jnp.float32)
    for _ in range(3):
        y = y * (1.5 - 0.5 * d * y * y)
    return jnp.where(d > 0.5, y, 0.0)


def _make_prep_kernel(N_pad, D, E, RT, NS):
    mesh = plsc.VectorSubcoreMesh(core_axis_name="c", subcore_axis_name="s")
    rows_e = E // EC
    base_r, extra = divmod(rows_e, NS)
    RB = 128
    n_rchunk = RT // RB

    @functools.partial(
        pl.kernel,
        mesh=mesh,
        compiler_params=pltpu.CompilerParams(needs_layout_passes=False),
        out_type=(
            jax.ShapeDtypeStruct((N_pad // 16, 16), jnp.float32),
            jax.ShapeDtypeStruct((N_pad, D), jnp.float32),
        ),
        scratch_types=[
            pltpu.VMEM((N_pad,), jnp.float32),
            pltpu.VMEM_SHARED((NS, N_pad), jnp.float32),
            pltpu.VMEM((EC,), jnp.int32),
            pltpu.VMEM((NS, RT), jnp.float32),
            pltpu.VMEM((RT // 16, 16), jnp.float32),
            pltpu.VMEM((RB, D), jnp.float32),
        ],
    )
    def prep(feat_hbm, dst2d_hbm, dinv_hbm, y0_hbm, deg_v, shr, dbuf, cbuf, dch, rbuf):
        cid = lax.axis_index("c")
        sid = lax.axis_index("s")
        work = cid == 0

        @pl.when(work)
        def _():
            @pl.loop(0, N_pad // 16)
            def _(i):
                deg_v[pl.ds(i * 16, 16)] = jnp.zeros((16,), jnp.float32)

            row0 = sid * base_r + jnp.minimum(sid, extra)
            trips = base_r + jnp.where(sid < extra, 1, 0)
            ones = jnp.ones((16,), jnp.float32)

            @pl.loop(0, trips)
            def _(j):
                pltpu.sync_copy(dst2d_hbm.at[row0 + j], dbuf)
                for g in range(EC // 16):
                    idx = dbuf[pl.ds(g * 16, 16)]
                    plsc.addupdate_scatter(deg_v, [idx], ones)

            pltpu.sync_copy(deg_v, shr.at[sid])

        plsc.subcore_barrier()

        @pl.when(work)
        def _():
            nb = sid * RT
            pltpu.sync_copy(shr.at[:, pl.ds(nb, RT)], cbuf)

            @pl.loop(0, RT // 16)
            def _(j):
                acc = jnp.zeros((16,), jnp.float32)
                for r in range(NS):
                    acc = acc + cbuf[r, pl.ds(j * 16, 16)]
                dch[j] = _rsqrt_newton(acc)

            pltpu.sync_copy(dch, dinv_hbm.at[pl.ds(sid * (RT // 16), RT // 16)])

            for c in range(n_rchunk):
                pltpu.sync_copy(feat_hbm.at[pl.ds(nb + c * RB, RB)], rbuf)

                @pl.loop(0, RB // 16)
                def _(g):
                    dvec = dch[c * (RB // 16) + g]
                    for r16 in range(16):
                        s = dvec[r16]
                        for j in range(D // 16):
                            row = g * 16 + r16
                            rbuf[row, pl.ds(j * 16, 16)] = (
                                rbuf[row, pl.ds(j * 16, 16)] * s)

                pltpu.sync_copy(rbuf, y0_hbm.at[pl.ds(nb + c * RB, RB)])

    return prep


def _make_hop_kernel(N_pad, D, E, RT, NS):
    mesh = plsc.VectorSubcoreMesh(core_axis_name="c", subcore_axis_name="s")
    rows_e = E // EC
    base_r, extra = divmod(rows_e, NS)
    RB = EC
    n_rchunk = RT // RB

    @functools.partial(
        pl.kernel,
        mesh=mesh,
        compiler_params=pltpu.CompilerParams(needs_layout_passes=False),
        out_type=(
            jax.ShapeDtypeStruct((N_pad, D), jnp.float32),
            jax.ShapeDtypeStruct((N_pad, D), jnp.float32),
        ),
        scratch_types=[
            pltpu.VMEM_SHARED((N_pad, D), jnp.float32),
            pltpu.VMEM((EC, D), jnp.float32),
            pltpu.VMEM((EC, D), jnp.float32),
            pltpu.VMEM((EC,), jnp.int32),
            pltpu.VMEM((EC,), jnp.int32),
            pltpu.VMEM((EC,), jnp.int32),
            pltpu.VMEM((EC,), jnp.int32),
            pltpu.VMEM((RT // 16, 16), jnp.float32),
            pltpu.VMEM((EC, D), jnp.float32),
            pltpu.SemaphoreType.DMA,
            pltpu.SemaphoreType.DMA,
        ],
    )
    def hop(y_hbm, src2d_hbm, dst2d_hbm, dinv_hbm, x_out, y_out,
            acc, bufA, bufB, srcA, srcB, dstA, dstB, dch, bufC,
            semA, semB):
        cid = lax.axis_index("c")
        sid = lax.axis_index("s")
        work = cid == 0

        @pl.when(work)
        def _():
            @pl.loop(0, RB)
            def _(r):
                for j in range(D // 16):
                    bufA[r, pl.ds(j * 16, 16)] = jnp.zeros((16,), jnp.float32)

            @pl.loop(0, n_rchunk)
            def _(c):
                pltpu.sync_copy(bufA, acc.at[pl.ds(sid * RT + c * RB, RB)])

        plsc.subcore_barrier()

        @pl.when(work)
        def _():
            row0 = sid * base_r + jnp.minimum(sid, extra)
            trips = base_r + jnp.where(sid < extra, 1, 0)
            trips2 = trips // 2

            pltpu.sync_copy(src2d_hbm.at[row0], srcA)
            pltpu.sync_copy(dst2d_hbm.at[row0], dstA)
            pltpu.async_copy(y_hbm.at[srcA], bufA, semA)

            @pl.loop(0, trips2)
            def _(i2):
                rB = row0 + 2 * i2 + 1
                pltpu.sync_copy(src2d_hbm.at[rB], srcB)
                pltpu.sync_copy(dst2d_hbm.at[rB], dstB)
                pltpu.async_copy(y_hbm.at[srcB], bufB, semB)
                pltpu.make_async_copy(y_hbm.at[srcA], bufA, semA).wait()
                pltpu.sync_copy(bufA, acc.at[dstA], add=True)

                @pl.when(2 * i2 + 2 < trips)
                def _():
                    rA = row0 + 2 * i2 + 2
                    pltpu.sync_copy(src2d_hbm.at[rA], srcA)
                    pltpu.sync_copy(dst2d_hbm.at[rA], dstA)
                    pltpu.async_copy(y_hbm.at[srcA], bufA, semA)

                pltpu.make_async_copy(y_hbm.at[srcB], bufB, semB).wait()
                pltpu.sync_copy(bufB, acc.at[dstB], add=True)

            @pl.when(trips % 2 == 1)
            def _():
                pltpu.make_async_copy(y_hbm.at[srcA], bufA, semA).wait()
                pltpu.sync_copy(bufA, acc.at[dstA], add=True)

        plsc.subcore_barrier()

        @pl.when(work)
        def _():
            nb = sid * RT
            pltpu.sync_copy(dinv_hbm.at[pl.ds(sid * (RT // 16), RT // 16)], dch)

            @pl.loop(0, n_rchunk)
            def _(c):
                r0 = nb + c * RB
                pltpu.sync_copy(acc.at[pl.ds(r0, RB)], bufA)

                @pl.loop(0, RB // 16)
                def _(g):
                    dvec = dch[c * (RB // 16) + g]
                    for r16 in range(16):
                        s = dvec[r16]
                        row = g * 16 + r16
                        for j in range(D // 16):
                            v = bufA[row, pl.ds(j * 16, 16)] * s
                            bufB[row, pl.ds(j * 16, 16)] = v
                            bufC[row, pl.ds(j * 16, 16)] = v * s

                pltpu.sync_copy(bufB, x_out.at[pl.ds(r0, RB)])
                pltpu.sync_copy(bufC, y_out.at[pl.ds(r0, RB)])

    return hop


def _mlp_body(*refs):
    xs_refs = refs[:-7]
    w1_ref, b1_ref, gs_ref, beta_ref, w2_ref, b2_ref, o_ref = refs[-7:]
    dn = (((1,), (1,)), ((), ()))
    h = lax.dot_general(xs_refs[0][...], w1_ref[0], dn,
                        preferred_element_type=jnp.float32)
    for k in range(1, len(xs_refs)):
        h = h + lax.dot_general(xs_refs[k][...], w1_ref[k], dn,
                                preferred_element_type=jnp.float32)
    h = h + b1_ref[...]
    h = jnp.where(h > 0, h, 0.2 * h)
    h = h * gs_ref[...] + beta_ref[...]
    o = lax.dot_general(h, w2_ref[...], dn,
                        preferred_element_type=jnp.float32) + b2_ref[...]
    o_ref[...] = o


def kernel(feature, edge_index, W1, b1, gamma, beta, W2, b2):
    N, D = feature.shape
    E = edge_index.shape[1]
    H, fan1 = W1.shape
    K = fan1 // D - 1
    O = W2.shape[0]
    NS = 16

    RT = ((N + NS - 1) // NS + 127) // 128 * 128
    N_pad = RT * NS

    feat_pad = jnp.pad(feature, ((0, N_pad - N), (0, 0)))
    src2d = edge_index[0].reshape(E // EC, EC)
    dst2d = edge_index[1].reshape(E // EC, EC)

    prep = _make_prep_kernel(N_pad, D, E, RT, NS)
    hop = _make_hop_kernel(N_pad, D, E, RT, NS)

    dinv, y = prep(feat_pad, dst2d)
    xs = [feat_pad]
    for _ in range(K):
        x_k, y = hop(y, src2d, dst2d, dinv)
        xs.append(x_k)

    W1r = W1.reshape(H, K + 1, D).transpose(1, 0, 2)
    gs = (gamma / jnp.sqrt(1.0 + 1e-5)).reshape(1, H)
    b1r = b1.reshape(1, H)
    betar = beta.reshape(1, H)
    b2r = b2.reshape(1, O)

    BM = 256
    grid = (N_pad // BM,)
    x_spec = pl.BlockSpec((BM, D), lambda i: (i, 0))
    out = pl.pallas_call(
        _mlp_body,
        grid=grid,
        in_specs=[x_spec] * (K + 1) + [
            pl.BlockSpec((K + 1, H, D), lambda i: (0, 0, 0)),
            pl.BlockSpec((1, H), lambda i: (0, 0)),
            pl.BlockSpec((1, H), lambda i: (0, 0)),
            pl.BlockSpec((1, H), lambda i: (0, 0)),
            pl.BlockSpec((O, H), lambda i: (0, 0)),
            pl.BlockSpec((1, O), lambda i: (0, 0)),
        ],
        out_specs=pl.BlockSpec((BM, O), lambda i: (i, 0)),
        out_shape=jax.ShapeDtypeStruct((N_pad, O), jnp.float32),
    )(*xs, W1r, b1r, gs, betar, W2, b2r)
    return out[:N]
